# Initial kernel scaffold; baseline (speedup 1.0000x reference)
import jax
import jax.numpy as jnp
from jax.experimental import pallas as pl


def kernel(x, table):
    raise NotImplementedError("write your pallas kernel here")


# speedup vs baseline: 3.0642x; 3.0642x over previous
import functools

import jax
import jax.numpy as jnp
from jax import lax
from jax.experimental import pallas as pl
from jax.experimental.pallas import tpu as pltpu
from jax.experimental.pallas import tpu_sc as plsc

EMBED_DIM = 128
CHUNK = 256


@functools.partial(jax.jit, static_argnames=())
def _gather_flat(idx_flat, table):
    n = idx_flat.shape[0]
    info = plsc.get_sparse_core_info()
    num_workers = info.num_cores * info.num_subcores
    per_worker = n // num_workers
    n_chunks = per_worker // CHUNK

    mesh = plsc.VectorSubcoreMesh(core_axis_name="c", subcore_axis_name="s")

    @functools.partial(
        pl.kernel,
        mesh=mesh,
        out_type=jax.ShapeDtypeStruct((n, EMBED_DIM), jnp.float32),
        scratch_types=[
            pltpu.VMEM((CHUNK,), jnp.int32),
            pltpu.VMEM((CHUNK, EMBED_DIM), jnp.float32),
            pltpu.SemaphoreType.DMA,
        ],
    )
    def k(idx_hbm, table_hbm, out_hbm, idx_v, rows_v, sem):
        wid = lax.axis_index("s") * info.num_cores + lax.axis_index("c")
        base = wid * per_worker

        def body(i, carry):
            off = base + i * CHUNK
            pltpu.sync_copy(idx_hbm.at[pl.ds(off, CHUNK)], idx_v)
            pltpu.async_copy(table_hbm.at[idx_v], rows_v, sem).wait()
            pltpu.sync_copy(rows_v, out_hbm.at[pl.ds(off, CHUNK)])
            return carry

        lax.fori_loop(0, n_chunks, body, 0)

    return k(idx_flat, table)


def kernel(x, table):
    b, s = x.shape
    idx_flat = x.reshape(-1).astype(jnp.int32)
    out = _gather_flat(idx_flat, table)
    return out.reshape(b, s, EMBED_DIM)


# speedup vs baseline: 3.2566x; 1.0628x over previous
import functools

import jax
import jax.numpy as jnp
from jax import lax
from jax.experimental import pallas as pl
from jax.experimental.pallas import tpu as pltpu
from jax.experimental.pallas import tpu_sc as plsc

EMBED_DIM = 128
CHUNK = 400
NBUF = 2


def _gather_flat(idx_flat, table):
    n = idx_flat.shape[0]
    info = plsc.get_sparse_core_info()
    num_workers = info.num_cores * info.num_subcores
    per_worker = n // num_workers
    n_chunks = per_worker // CHUNK
    n_groups = n_chunks // NBUF

    mesh = plsc.VectorSubcoreMesh(core_axis_name="c", subcore_axis_name="s")

    @functools.partial(
        pl.kernel,
        mesh=mesh,
        out_type=jax.ShapeDtypeStruct((n, EMBED_DIM), jnp.float32),
        scratch_types=[
            pltpu.VMEM((per_worker,), jnp.int32),
            pltpu.VMEM((CHUNK, EMBED_DIM), jnp.float32),
            pltpu.VMEM((CHUNK, EMBED_DIM), jnp.float32),
            pltpu.SemaphoreType.DMA,
            pltpu.SemaphoreType.DMA,
            pltpu.SemaphoreType.DMA,
            pltpu.SemaphoreType.DMA,
        ],
    )
    def k(idx_hbm, table_hbm, out_hbm, idx_v, buf0, buf1,
          gsem0, gsem1, wsem0, wsem1):
        bufs = (buf0, buf1)
        gsems = (gsem0, gsem1)
        wsems = (wsem0, wsem1)
        wid = lax.axis_index("s") * info.num_cores + lax.axis_index("c")
        base = wid * per_worker

        pltpu.sync_copy(idx_hbm.at[pl.ds(base, per_worker)], idx_v)

        for b in range(NBUF):
            pltpu.async_copy(
                table_hbm.at[idx_v.at[pl.ds(b * CHUNK, CHUNK)]],
                bufs[b], gsems[b])

        def body(g, carry):
            c0 = g * NBUF
            for b in range(NBUF):
                off = base + (c0 + b) * CHUNK
                dst = out_hbm.at[pl.ds(off, CHUNK)]
                pltpu.make_async_copy(dst, bufs[b], gsems[b]).wait()
                pltpu.async_copy(bufs[b], dst, wsems[b])
            for b in range(NBUF):
                off = base + (c0 + b) * CHUNK
                pltpu.make_async_copy(
                    bufs[b], out_hbm.at[pl.ds(off, CHUNK)], wsems[b]).wait()

                @pl.when(c0 + b + NBUF < n_chunks)
                def _():
                    pltpu.async_copy(
                        table_hbm.at[idx_v.at[pl.ds((c0 + b + NBUF) * CHUNK, CHUNK)]],
                        bufs[b], gsems[b])
            return carry

        lax.fori_loop(0, n_groups, body, 0)

    return k(idx_flat, table)


def kernel(x, table):
    b, s = x.shape
    idx_flat = x.reshape(-1).astype(jnp.int32)
    out = _gather_flat(idx_flat, table)
    return out.reshape(b, s, EMBED_DIM)


# speedup vs baseline: 5.6680x; 1.7405x over previous
import functools

import jax
import jax.numpy as jnp
from jax import lax
from jax.experimental import pallas as pl
from jax.experimental.pallas import tpu as pltpu
from jax.experimental.pallas import tpu_sc as plsc

EMBED_DIM = 128
SLAB = 50
SLABS_PER_CHUNK = 8
CHUNK = SLAB * SLABS_PER_CHUNK
NBUF = 2


def _gather(idx_flat, table, n_outer):
    n = idx_flat.shape[0]
    info = plsc.get_sparse_core_info()
    num_workers = info.num_cores * info.num_subcores
    per_worker = n // num_workers
    slabs_per_worker = n_outer // num_workers
    n_chunks = per_worker // CHUNK
    n_groups = n_chunks // NBUF

    mesh = plsc.VectorSubcoreMesh(core_axis_name="c", subcore_axis_name="s")

    @functools.partial(
        pl.kernel,
        mesh=mesh,
        out_type=jax.ShapeDtypeStruct((n_outer, SLAB, EMBED_DIM), jnp.float32),
        scratch_types=[
            pltpu.VMEM((per_worker,), jnp.int32),
            pltpu.VMEM((CHUNK, EMBED_DIM), jnp.float32),
            pltpu.VMEM((CHUNK, EMBED_DIM), jnp.float32),
            pltpu.SemaphoreType.DMA,
            pltpu.SemaphoreType.DMA,
            pltpu.SemaphoreType.DMA,
            pltpu.SemaphoreType.DMA,
        ],
    )
    def k(idx_hbm, table_hbm, out_hbm, idx_v, buf0, buf1,
          gsem0, gsem1, wsem0, wsem1):
        bufs = (buf0, buf1)
        gsems = (gsem0, gsem1)
        wsems = (wsem0, wsem1)
        wid = lax.axis_index("s") * info.num_cores + lax.axis_index("c")
        base = wid * per_worker
        slab_base = wid * slabs_per_worker
        dummy = table_hbm.at[pl.ds(0, CHUNK)]

        pltpu.sync_copy(idx_hbm.at[pl.ds(base, per_worker)], idx_v)

        for b in range(NBUF):
            pltpu.async_copy(
                table_hbm.at[idx_v.at[pl.ds(b * CHUNK, CHUNK)]],
                bufs[b], gsems[b])

        def body(g, carry):
            c0 = g * NBUF
            for b in range(NBUF):
                c = c0 + b
                pltpu.make_async_copy(dummy, bufs[b], gsems[b]).wait()
                for s in range(SLABS_PER_CHUNK):
                    pltpu.async_copy(
                        bufs[b].at[pl.ds(s * SLAB, SLAB)],
                        out_hbm.at[slab_base + c * SLABS_PER_CHUNK + s],
                        wsems[b])
            for b in range(NBUF):
                c = c0 + b
                pltpu.make_async_copy(bufs[b], dummy, wsems[b]).wait()

                @pl.when(c + NBUF < n_chunks)
                def _():
                    pltpu.async_copy(
                        table_hbm.at[idx_v.at[pl.ds((c + NBUF) * CHUNK, CHUNK)]],
                        bufs[b], gsems[b])
            return carry

        lax.fori_loop(0, n_groups, body, 0)

    return k(idx_flat, table)


def kernel(x, table):
    b, s = x.shape
    idx_flat = x.reshape(-1).astype(jnp.int32)
    return _gather(idx_flat, table, b)


# speedup vs baseline: 9.7403x; 1.7185x over previous
import functools

import jax
import jax.numpy as jnp
from jax import lax
from jax.experimental import pallas as pl
from jax.experimental.pallas import tpu as pltpu
from jax.experimental.pallas import tpu_sc as plsc

EMBED_DIM = 128
CHUNK = 400
NBUF = 2


def _gather_flat(idx_flat, table):
    n = idx_flat.shape[0]
    info = plsc.get_sparse_core_info()
    num_workers = info.num_cores * info.num_subcores
    per_worker = n // num_workers
    n_chunks = per_worker // CHUNK
    n_groups = n_chunks // NBUF

    mesh = plsc.VectorSubcoreMesh(core_axis_name="c", subcore_axis_name="s")

    @functools.partial(
        pl.kernel,
        mesh=mesh,
        out_type=jax.ShapeDtypeStruct((n, EMBED_DIM), jnp.float32),
        scratch_types=[
            pltpu.VMEM((per_worker,), jnp.int32),
            pltpu.VMEM((CHUNK, EMBED_DIM), jnp.float32),
            pltpu.VMEM((CHUNK, EMBED_DIM), jnp.float32),
            pltpu.SemaphoreType.DMA,
            pltpu.SemaphoreType.DMA,
            pltpu.SemaphoreType.DMA,
            pltpu.SemaphoreType.DMA,
        ],
    )
    def k(idx_hbm, table_hbm, out_hbm, idx_v, buf0, buf1,
          gsem0, gsem1, wsem0, wsem1):
        bufs = (buf0, buf1)
        gsems = (gsem0, gsem1)
        wsems = (wsem0, wsem1)
        wid = lax.axis_index("s") * info.num_cores + lax.axis_index("c")
        base = wid * per_worker

        pltpu.sync_copy(idx_hbm.at[pl.ds(base, per_worker)], idx_v)

        for b in range(NBUF):
            pltpu.async_copy(
                table_hbm.at[idx_v.at[pl.ds(b * CHUNK, CHUNK)]],
                bufs[b], gsems[b])

        def body(g, carry):
            c0 = g * NBUF
            for b in range(NBUF):
                off = base + (c0 + b) * CHUNK
                dst = out_hbm.at[pl.ds(off, CHUNK)]
                pltpu.make_async_copy(dst, bufs[b], gsems[b]).wait()
                pltpu.async_copy(bufs[b], dst, wsems[b])
            for b in range(NBUF):
                off = base + (c0 + b) * CHUNK
                pltpu.make_async_copy(
                    bufs[b], out_hbm.at[pl.ds(off, CHUNK)], wsems[b]).wait()

                @pl.when(c0 + b + NBUF < n_chunks)
                def _():
                    pltpu.async_copy(
                        table_hbm.at[idx_v.at[pl.ds((c0 + b + NBUF) * CHUNK, CHUNK)]],
                        bufs[b], gsems[b])
            return carry

        lax.fori_loop(0, n_groups, body, 0)

    return k(idx_flat, table)


def kernel(x, table):
    b, s = x.shape
    idx_flat = x.T.reshape(-1).astype(jnp.int32)
    out = _gather_flat(idx_flat, table)
    return out.reshape(s, b, EMBED_DIM).transpose(1, 0, 2)


# speedup vs baseline: 9.9539x; 1.0219x over previous
import functools

import jax
import jax.numpy as jnp
from jax import lax
from jax.experimental import pallas as pl
from jax.experimental.pallas import tpu as pltpu
from jax.experimental.pallas import tpu_sc as plsc

EMBED_DIM = 128
CHUNK = 200
NBUF = 4


def _gather_flat(idx_flat, table):
    n = idx_flat.shape[0]
    info = plsc.get_sparse_core_info()
    num_workers = info.num_cores * info.num_subcores
    per_worker = n // num_workers
    n_chunks = per_worker // CHUNK
    n_groups = n_chunks // NBUF

    mesh = plsc.VectorSubcoreMesh(core_axis_name="c", subcore_axis_name="s")

    @functools.partial(
        pl.kernel,
        mesh=mesh,
        out_type=jax.ShapeDtypeStruct((n, EMBED_DIM), jnp.float32),
        scratch_types=[
            pltpu.VMEM((per_worker,), jnp.int32),
        ] + [pltpu.VMEM((CHUNK, EMBED_DIM), jnp.float32)] * NBUF
          + [pltpu.SemaphoreType.DMA] * (2 * NBUF),
    )
    def k(idx_hbm, table_hbm, out_hbm, idx_v, *rest):
        bufs = rest[:NBUF]
        gsems = rest[NBUF:2 * NBUF]
        wsems = rest[2 * NBUF:]
        wid = lax.axis_index("s") * info.num_cores + lax.axis_index("c")
        base = wid * per_worker

        pltpu.sync_copy(idx_hbm.at[pl.ds(base, per_worker)], idx_v)

        for b in range(NBUF):
            pltpu.async_copy(
                table_hbm.at[idx_v.at[pl.ds(b * CHUNK, CHUNK)]],
                bufs[b], gsems[b])

        def body(g, carry):
            c0 = g * NBUF
            for b in range(NBUF):
                off = base + (c0 + b) * CHUNK
                dst = out_hbm.at[pl.ds(off, CHUNK)]
                pltpu.make_async_copy(dst, bufs[b], gsems[b]).wait()
                pltpu.async_copy(bufs[b], dst, wsems[b])
            for b in range(NBUF):
                off = base + (c0 + b) * CHUNK
                pltpu.make_async_copy(
                    bufs[b], out_hbm.at[pl.ds(off, CHUNK)], wsems[b]).wait()

                @pl.when(c0 + b + NBUF < n_chunks)
                def _():
                    pltpu.async_copy(
                        table_hbm.at[idx_v.at[pl.ds((c0 + b + NBUF) * CHUNK, CHUNK)]],
                        bufs[b], gsems[b])
            return carry

        lax.fori_loop(0, n_groups, body, 0)

    return k(idx_flat, table)


def kernel(x, table):
    b, s = x.shape
    idx_flat = x.T.reshape(-1).astype(jnp.int32)
    out = _gather_flat(idx_flat, table)
    return out.reshape(s, b, EMBED_DIM).transpose(1, 0, 2)


# speedup vs baseline: 10.1766x; 1.0224x over previous
import functools

import jax
import jax.numpy as jnp
from jax import lax
from jax.experimental import pallas as pl
from jax.experimental.pallas import tpu as pltpu
from jax.experimental.pallas import tpu_sc as plsc

EMBED_DIM = 128
CHUNK = 128
NBUF = 5


def _gather_flat(idx_flat, table):
    n = idx_flat.shape[0]
    info = plsc.get_sparse_core_info()
    num_workers = info.num_cores * info.num_subcores
    per_worker = n // num_workers
    n_chunks = per_worker // CHUNK
    n_groups = n_chunks // NBUF

    mesh = plsc.VectorSubcoreMesh(core_axis_name="c", subcore_axis_name="s")

    @functools.partial(
        pl.kernel,
        mesh=mesh,
        out_type=jax.ShapeDtypeStruct((n, EMBED_DIM), jnp.float32),
        scratch_types=[
            pltpu.VMEM((per_worker,), jnp.int32),
        ] + [pltpu.VMEM((CHUNK, EMBED_DIM), jnp.float32)] * NBUF
          + [pltpu.SemaphoreType.DMA] * (2 * NBUF),
    )
    def k(idx_hbm, table_hbm, out_hbm, idx_v, *rest):
        bufs = rest[:NBUF]
        gsems = rest[NBUF:2 * NBUF]
        wsems = rest[2 * NBUF:]
        wid = lax.axis_index("s") * info.num_cores + lax.axis_index("c")
        base = wid * per_worker

        pltpu.sync_copy(idx_hbm.at[pl.ds(base, per_worker)], idx_v)

        for b in range(NBUF):
            pltpu.async_copy(
                table_hbm.at[idx_v.at[pl.ds(b * CHUNK, CHUNK)]],
                bufs[b], gsems[b])

        def body(g, carry):
            c0 = g * NBUF
            for b in range(NBUF):
                off = base + (c0 + b) * CHUNK
                dst = out_hbm.at[pl.ds(off, CHUNK)]
                pltpu.make_async_copy(dst, bufs[b], gsems[b]).wait()
                pltpu.async_copy(bufs[b], dst, wsems[b])
            for b in range(NBUF):
                off = base + (c0 + b) * CHUNK
                pltpu.make_async_copy(
                    bufs[b], out_hbm.at[pl.ds(off, CHUNK)], wsems[b]).wait()

                @pl.when(c0 + b + NBUF < n_chunks)
                def _():
                    pltpu.async_copy(
                        table_hbm.at[idx_v.at[pl.ds((c0 + b + NBUF) * CHUNK, CHUNK)]],
                        bufs[b], gsems[b])
            return carry

        lax.fori_loop(0, n_groups, body, 0)

    return k(idx_flat, table)


def kernel(x, table):
    b, s = x.shape
    idx_flat = x.T.reshape(-1).astype(jnp.int32)
    out = _gather_flat(idx_flat, table)
    return out.reshape(s, b, EMBED_DIM).transpose(1, 0, 2)


# speedup vs baseline: 10.2429x; 1.0065x over previous
import functools

import jax
import jax.numpy as jnp
from jax import lax
from jax.experimental import pallas as pl
from jax.experimental.pallas import tpu as pltpu
from jax.experimental.pallas import tpu_sc as plsc

EMBED_DIM = 128
CHUNK = 80
NBUF = 10


def _gather_flat(idx_flat, table):
    n = idx_flat.shape[0]
    info = plsc.get_sparse_core_info()
    num_workers = info.num_cores * info.num_subcores
    per_worker = n // num_workers
    n_chunks = per_worker // CHUNK
    n_groups = n_chunks // NBUF

    mesh = plsc.VectorSubcoreMesh(core_axis_name="c", subcore_axis_name="s")

    @functools.partial(
        pl.kernel,
        mesh=mesh,
        out_type=jax.ShapeDtypeStruct((n, EMBED_DIM), jnp.float32),
        scratch_types=[
            pltpu.VMEM((per_worker,), jnp.int32),
        ] + [pltpu.VMEM((CHUNK, EMBED_DIM), jnp.float32)] * NBUF
          + [pltpu.SemaphoreType.DMA] * (2 * NBUF),
    )
    def k(idx_hbm, table_hbm, out_hbm, idx_v, *rest):
        bufs = rest[:NBUF]
        gsems = rest[NBUF:2 * NBUF]
        wsems = rest[2 * NBUF:]
        wid = lax.axis_index("s") * info.num_cores + lax.axis_index("c")
        base = wid * per_worker

        pltpu.sync_copy(idx_hbm.at[pl.ds(base, per_worker)], idx_v)

        for b in range(NBUF):
            pltpu.async_copy(
                table_hbm.at[idx_v.at[pl.ds(b * CHUNK, CHUNK)]],
                bufs[b], gsems[b])

        def body(g, carry):
            c0 = g * NBUF
            for b in range(NBUF):
                off = base + (c0 + b) * CHUNK
                dst = out_hbm.at[pl.ds(off, CHUNK)]
                pltpu.make_async_copy(dst, bufs[b], gsems[b]).wait()
                pltpu.async_copy(bufs[b], dst, wsems[b])
            for b in range(NBUF):
                off = base + (c0 + b) * CHUNK
                pltpu.make_async_copy(
                    bufs[b], out_hbm.at[pl.ds(off, CHUNK)], wsems[b]).wait()

                @pl.when(c0 + b + NBUF < n_chunks)
                def _():
                    pltpu.async_copy(
                        table_hbm.at[idx_v.at[pl.ds((c0 + b + NBUF) * CHUNK, CHUNK)]],
                        bufs[b], gsems[b])
            return carry

        lax.fori_loop(0, n_groups, body, 0)

    return k(idx_flat, table)


def kernel(x, table):
    b, s = x.shape
    idx_flat = x.T.reshape(-1).astype(jnp.int32)
    out = _gather_flat(idx_flat, table)
    return out.reshape(s, b, EMBED_DIM).transpose(1, 0, 2)


# speedup vs baseline: 10.3051x; 1.0061x over previous
import functools

import jax
import jax.numpy as jnp
from jax import lax
from jax.experimental import pallas as pl
from jax.experimental.pallas import tpu as pltpu
from jax.experimental.pallas import tpu_sc as plsc

EMBED_DIM = 128
CHUNK = 64
NBUF = 10


def _gather_flat(idx_flat, table):
    n = idx_flat.shape[0]
    info = plsc.get_sparse_core_info()
    num_workers = info.num_cores * info.num_subcores
    per_worker = n // num_workers
    n_chunks = per_worker // CHUNK
    n_groups = n_chunks // NBUF

    mesh = plsc.VectorSubcoreMesh(core_axis_name="c", subcore_axis_name="s")

    @functools.partial(
        pl.kernel,
        mesh=mesh,
        out_type=jax.ShapeDtypeStruct((n, EMBED_DIM), jnp.float32),
        scratch_types=[
            pltpu.VMEM((per_worker,), jnp.int32),
        ] + [pltpu.VMEM((CHUNK, EMBED_DIM), jnp.float32)] * NBUF
          + [pltpu.SemaphoreType.DMA] * (2 * NBUF),
    )
    def k(idx_hbm, table_hbm, out_hbm, idx_v, *rest):
        bufs = rest[:NBUF]
        gsems = rest[NBUF:2 * NBUF]
        wsems = rest[2 * NBUF:]
        wid = lax.axis_index("s") * info.num_cores + lax.axis_index("c")
        base = wid * per_worker

        pltpu.sync_copy(idx_hbm.at[pl.ds(base, per_worker)], idx_v)

        for b in range(NBUF):
            pltpu.async_copy(
                table_hbm.at[idx_v.at[pl.ds(b * CHUNK, CHUNK)]],
                bufs[b], gsems[b])

        def body(g, carry):
            c0 = g * NBUF
            for b in range(NBUF):
                off = base + (c0 + b) * CHUNK
                dst = out_hbm.at[pl.ds(off, CHUNK)]
                pltpu.make_async_copy(dst, bufs[b], gsems[b]).wait()
                pltpu.async_copy(bufs[b], dst, wsems[b])
            for b in range(NBUF):
                off = base + (c0 + b) * CHUNK
                pltpu.make_async_copy(
                    bufs[b], out_hbm.at[pl.ds(off, CHUNK)], wsems[b]).wait()

                @pl.when(c0 + b + NBUF < n_chunks)
                def _():
                    pltpu.async_copy(
                        table_hbm.at[idx_v.at[pl.ds((c0 + b + NBUF) * CHUNK, CHUNK)]],
                        bufs[b], gsems[b])
            return carry

        lax.fori_loop(0, n_groups, body, 0)

    return k(idx_flat, table)


def kernel(x, table):
    b, s = x.shape
    idx_flat = x.T.reshape(-1).astype(jnp.int32)
    out = _gather_flat(idx_flat, table)
    return out.reshape(s, b, EMBED_DIM).transpose(1, 0, 2)


# speedup vs baseline: 10.3152x; 1.0010x over previous
import functools

import jax
import jax.numpy as jnp
from jax import lax
from jax.experimental import pallas as pl
from jax.experimental.pallas import tpu as pltpu
from jax.experimental.pallas import tpu_sc as plsc

EMBED_DIM = 128
CHUNK = 64
NBUF = 10
SUB = 2


def _gather_flat(idx_flat, table):
    n = idx_flat.shape[0]
    info = plsc.get_sparse_core_info()
    num_workers = info.num_cores * info.num_subcores
    per_worker = n // num_workers
    n_chunks = per_worker // CHUNK
    n_groups = n_chunks // NBUF

    mesh = plsc.VectorSubcoreMesh(core_axis_name="c", subcore_axis_name="s")

    @functools.partial(
        pl.kernel,
        mesh=mesh,
        out_type=jax.ShapeDtypeStruct((n, EMBED_DIM), jnp.float32),
        scratch_types=[
            pltpu.VMEM((per_worker,), jnp.int32),
        ] + [pltpu.VMEM((CHUNK, EMBED_DIM), jnp.float32)] * NBUF
          + [pltpu.SemaphoreType.DMA] * (2 * NBUF),
    )
    def k(idx_hbm, table_hbm, out_hbm, idx_v, *rest):
        bufs = rest[:NBUF]
        gsems = rest[NBUF:2 * NBUF]
        wsems = rest[2 * NBUF:]
        wid = lax.axis_index("s") * info.num_cores + lax.axis_index("c")
        base = wid * per_worker

        pltpu.sync_copy(idx_hbm.at[pl.ds(base, per_worker)], idx_v)

        sub = CHUNK // SUB

        def fire_gather(c, b):
            for u in range(SUB):
                pltpu.async_copy(
                    table_hbm.at[idx_v.at[pl.ds(c * CHUNK + u * sub, sub)]],
                    bufs[b].at[pl.ds(u * sub, sub)], gsems[b])

        for b in range(NBUF):
            fire_gather(b, b)

        def body(g, carry):
            c0 = g * NBUF
            for b in range(NBUF):
                off = base + (c0 + b) * CHUNK
                dst = out_hbm.at[pl.ds(off, CHUNK)]
                pltpu.make_async_copy(dst, bufs[b], gsems[b]).wait()
                pltpu.async_copy(bufs[b], dst, wsems[b])
            for b in range(NBUF):
                off = base + (c0 + b) * CHUNK
                pltpu.make_async_copy(
                    bufs[b], out_hbm.at[pl.ds(off, CHUNK)], wsems[b]).wait()

                @pl.when(c0 + b + NBUF < n_chunks)
                def _():
                    fire_gather(c0 + b + NBUF, b)
            return carry

        lax.fori_loop(0, n_groups, body, 0)

    return k(idx_flat, table)


def kernel(x, table):
    b, s = x.shape
    idx_flat = x.T.reshape(-1).astype(jnp.int32)
    out = _gather_flat(idx_flat, table)
    return out.reshape(s, b, EMBED_DIM).transpose(1, 0, 2)


# speedup vs baseline: 10.4325x; 1.0114x over previous
import functools

import jax
import jax.numpy as jnp
from jax import lax
from jax.experimental import pallas as pl
from jax.experimental.pallas import tpu as pltpu
from jax.experimental.pallas import tpu_sc as plsc

EMBED_DIM = 128
NBUF = 5


def _gather_t(idx_t, table):
    n_rows, n_cols = idx_t.shape
    info = plsc.get_sparse_core_info()
    num_workers = info.num_cores * info.num_subcores
    cols_per_w = n_cols // num_workers
    n_groups = n_rows // NBUF

    mesh = plsc.VectorSubcoreMesh(core_axis_name="c", subcore_axis_name="s")

    @functools.partial(
        pl.kernel,
        mesh=mesh,
        out_type=jax.ShapeDtypeStruct((n_rows * n_cols, EMBED_DIM), jnp.float32),
        scratch_types=[
            pltpu.VMEM((n_rows, cols_per_w), jnp.int32),
        ] + [pltpu.VMEM((cols_per_w, EMBED_DIM), jnp.float32)] * NBUF
          + [pltpu.SemaphoreType.DMA] * (2 * NBUF),
    )
    def k(idx_hbm, table_hbm, out_hbm, idx_v, *rest):
        bufs = rest[:NBUF]
        gsems = rest[NBUF:2 * NBUF]
        wsems = rest[2 * NBUF:]
        wid = lax.axis_index("s") * info.num_cores + lax.axis_index("c")
        col0 = wid * cols_per_w

        pltpu.sync_copy(idx_hbm.at[:, pl.ds(col0, cols_per_w)], idx_v)

        for b in range(NBUF):
            pltpu.async_copy(table_hbm.at[idx_v.at[b]], bufs[b], gsems[b])

        def body(g, carry):
            j0 = g * NBUF
            for b in range(NBUF):
                dst = out_hbm.at[pl.ds((j0 + b) * n_cols + col0, cols_per_w)]
                pltpu.make_async_copy(dst, bufs[b], gsems[b]).wait()
                pltpu.async_copy(bufs[b], dst, wsems[b])
            for b in range(NBUF):
                dst = out_hbm.at[pl.ds((j0 + b) * n_cols + col0, cols_per_w)]
                pltpu.make_async_copy(bufs[b], dst, wsems[b]).wait()

                @pl.when(j0 + b + NBUF < n_rows)
                def _():
                    pltpu.async_copy(
                        table_hbm.at[idx_v.at[j0 + b + NBUF]], bufs[b], gsems[b])
            return carry

        lax.fori_loop(0, n_groups, body, 0)

    return k(idx_t, table)


def kernel(x, table):
    b, s = x.shape
    idx_t = x.T.astype(jnp.int32)
    out = _gather_t(idx_t, table)
    return out.reshape(s, b, EMBED_DIM).transpose(1, 0, 2)


# speedup vs baseline: 10.6758x; 1.0233x over previous
import functools

import jax
import jax.numpy as jnp
from jax import lax
from jax.experimental import pallas as pl
from jax.experimental.pallas import tpu as pltpu
from jax.experimental.pallas import tpu_sc as plsc

EMBED_DIM = 128
NBUF = 10


def _gather_t(idx_t, table):
    n_rows, n_cols = idx_t.shape
    info = plsc.get_sparse_core_info()
    num_workers = info.num_cores * info.num_subcores
    cols_per_w = n_cols // num_workers
    half = cols_per_w // 2
    n_units = n_rows * 2
    n_groups = n_units // NBUF

    mesh = plsc.VectorSubcoreMesh(core_axis_name="c", subcore_axis_name="s")

    @functools.partial(
        pl.kernel,
        mesh=mesh,
        out_type=jax.ShapeDtypeStruct((n_rows * n_cols, EMBED_DIM), jnp.float32),
        scratch_types=[
            pltpu.VMEM((n_rows, cols_per_w), jnp.int32),
        ] + [pltpu.VMEM((half, EMBED_DIM), jnp.float32)] * NBUF
          + [pltpu.SemaphoreType.DMA] * (2 * NBUF),
    )
    def k(idx_hbm, table_hbm, out_hbm, idx_v, *rest):
        bufs = rest[:NBUF]
        gsems = rest[NBUF:2 * NBUF]
        wsems = rest[2 * NBUF:]
        wid = lax.axis_index("s") * info.num_cores + lax.axis_index("c")
        col0 = wid * cols_per_w

        pltpu.sync_copy(idx_hbm.at[:, pl.ds(col0, cols_per_w)], idx_v)

        def unit_dst(u):
            return out_hbm.at[
                pl.ds((u // 2) * n_cols + col0 + (u % 2) * half, half)]

        def fire_gather(u, b):
            pltpu.async_copy(
                table_hbm.at[idx_v.at[u // 2, pl.ds((u % 2) * half, half)]],
                bufs[b], gsems[b])

        for b in range(NBUF):
            fire_gather(b, b)

        def body(g, carry):
            u0 = g * NBUF
            for b in range(NBUF):
                dst = unit_dst(u0 + b)
                pltpu.make_async_copy(dst, bufs[b], gsems[b]).wait()
                pltpu.async_copy(bufs[b], dst, wsems[b])
            for b in range(NBUF):
                pltpu.make_async_copy(bufs[b], unit_dst(u0 + b), wsems[b]).wait()

                @pl.when(u0 + b + NBUF < n_units)
                def _():
                    fire_gather(u0 + b + NBUF, b)
            return carry

        lax.fori_loop(0, n_groups, body, 0)

    return k(idx_t, table)


def kernel(x, table):
    b, s = x.shape
    idx_t = x.T.astype(jnp.int32)
    out = _gather_t(idx_t, table)
    return out.reshape(s, b, EMBED_DIM).transpose(1, 0, 2)
